# baseline (speedup 1.0000x reference)
import functools

import jax
import jax.numpy as jnp
from jax import lax
from jax.experimental import pallas as pl
from jax.experimental.pallas import tpu as pltpu
from jax.experimental.pallas import tpu_sc as plsc

N = 10000
E = 320000
H = 128
F = 64
R = 50
L = 6
CUT = 5.0

NPAD = 10240
NTILE = 32
EPT = E // NTILE
SUB = 100
SB = 200
NSUPER = EPT // SB
ROWS_PER_TILE = NPAD // 16
ZROWS = 128


def _pack_bf16_pairs(w):
    lo = w[:, :F // 2].astype(jnp.bfloat16).astype(jnp.float32)
    hi = w[:, F // 2:].astype(jnp.bfloat16).astype(jnp.float32)
    ulo = lax.shift_right_logical(lax.bitcast_convert_type(lo, jnp.int32), 16)
    uhi = lax.bitcast_convert_type(hi, jnp.int32) & jnp.int32(-65536)
    return ulo | uhi


def _embed_body(z_ref, emb_ref, w1_ref, x_ref, xh_ref):
    zv = z_ref[0, 0, :]
    col = lax.broadcasted_iota(jnp.int32, (zv.shape[0], H), 1)
    oh = (zv[:, None] == col).astype(jnp.float32)
    x = jnp.dot(oh, emb_ref[...], preferred_element_type=jnp.float32)
    x_ref[...] = x
    xh_ref[...] = _pack_bf16_pairs(
        lax.dot_general(x.astype(jnp.bfloat16),
                        w1_ref[...].astype(jnp.bfloat16),
                        (((1,), (1,)), ((), ())),
                        preferred_element_type=jnp.float32))


def _embed(z, embedding, w1l):
    z3 = z.astype(jnp.int32).reshape(10, 1, N // 10)
    emb_pad = jnp.zeros((H, H), jnp.float32).at[:100, :].set(embedding)
    return pl.pallas_call(
        _embed_body,
        grid=(10,),
        in_specs=[
            pl.BlockSpec((1, 1, N // 10), lambda i: (i, 0, 0)),
            pl.BlockSpec((H, H), lambda i: (0, 0)),
            pl.BlockSpec((F, H), lambda i: (0, 0)),
        ],
        out_specs=[
            pl.BlockSpec((N // 10, H), lambda i: (i, 0)),
            pl.BlockSpec((N // 10, F // 2), lambda i: (i, 0)),
        ],
        out_shape=[
            jax.ShapeDtypeStruct((N, H), jnp.float32),
            jax.ShapeDtypeStruct((N, F // 2), jnp.int32),
        ],
    )(z3, emb_pad, w1l)


KTAB = 16128
CT = 4032


def _table_body(means_ref, betas_ref, w1_ref, b1_ref, w2_ref, b2_ref,
                out_ref):
    base = pl.program_id(1) * CT
    j = lax.broadcasted_iota(jnp.int32, (CT,), 0) + base
    d = j.astype(jnp.float32) * (CUT / KTAB)
    t = jnp.exp(-d)
    m = means_ref[0]
    be = betas_ref[0]
    ea = jnp.exp(-be[None, :] * (t[:, None] - m[None, :]) ** 2)
    h = lax.dot_general(ea.astype(jnp.bfloat16), w1_ref[0].astype(jnp.bfloat16),
                        (((1,), (1,)), ((), ())),
                        preferred_element_type=jnp.float32)
    h = h + b1_ref[0, 0][None, :]
    h = h * jax.nn.sigmoid(h)
    w = lax.dot_general(h.astype(jnp.bfloat16), w2_ref[0].astype(jnp.bfloat16),
                        (((1,), (1,)), ((), ())),
                        preferred_element_type=jnp.float32)
    w = w + b2_ref[0, 0][None, :]
    c = 0.5 * (jnp.cos(d * (jnp.pi / CUT)) + 1.0) * (d < CUT).astype(jnp.float32)
    out_ref[0] = _pack_bf16_pairs(w * c[:, None])


def _tables(rbf_means, rbf_betas, mlp_w1, mlp_b1, mlp_w2, mlp_b2):
    return pl.pallas_call(
        _table_body,
        grid=(L, KTAB // CT),
        in_specs=[
            pl.BlockSpec((1, R), lambda l, i: (0, 0)),
            pl.BlockSpec((1, R), lambda l, i: (0, 0)),
            pl.BlockSpec((1, F, R), lambda l, i: (l, 0, 0)),
            pl.BlockSpec((1, 1, F), lambda l, i: (l, 0, 0)),
            pl.BlockSpec((1, F, F), lambda l, i: (l, 0, 0)),
            pl.BlockSpec((1, 1, F), lambda l, i: (l, 0, 0)),
        ],
        out_specs=pl.BlockSpec((1, CT, F // 2), lambda l, i: (l, i, 0)),
        out_shape=jax.ShapeDtypeStruct((L, KTAB, F // 2), jnp.int32),
    )(rbf_means.reshape(1, R), rbf_betas.reshape(1, R),
      mlp_w1, mlp_b1.reshape(L, 1, F), mlp_w2, mlp_b2.reshape(L, 1, F))


NB = 100
CE = E // NB


def _tidx_body(ew_ref, out_ref):
    d = ew_ref[0, 0, :]
    out_ref[0, 0, :] = (d * (KTAB / CUT) + 0.5).astype(jnp.int32)


def _tidx(edge_weight):
    return pl.pallas_call(
        _tidx_body,
        grid=(NB,),
        in_specs=[pl.BlockSpec((1, 1, CE), lambda i: (i, 0, 0))],
        out_specs=pl.BlockSpec((1, 1, CE), lambda i: (i, 0, 0)),
        out_shape=jax.ShapeDtypeStruct((NB, 1, CE), jnp.int32),
    )(edge_weight.reshape(NB, 1, CE))


def _unpack_bf16_pairs(p):
    lo = lax.bitcast_convert_type(jnp.left_shift(p, 16), jnp.float32)
    hi = lax.bitcast_convert_type(p & jnp.int32(-65536), jnp.float32)
    return jnp.concatenate([lo, hi], axis=1)


def _tail_body(x_ref, parts_ref, w2_ref, b2_ref, wl_ref, bl_ref, xo_ref):
    agg = (_unpack_bf16_pairs(parts_ref[0]) +
           _unpack_bf16_pairs(parts_ref[1]))
    y = lax.dot_general(agg.astype(jnp.bfloat16),
                        w2_ref[...].astype(jnp.bfloat16),
                        (((1,), (1,)), ((), ())),
                        preferred_element_type=jnp.float32)
    y = y + b2_ref[0][None, :]
    y = y * jax.nn.sigmoid(y)
    y = lax.dot_general(y.astype(jnp.bfloat16),
                        wl_ref[...].astype(jnp.bfloat16),
                        (((1,), (1,)), ((), ())),
                        preferred_element_type=jnp.float32)
    y = y + bl_ref[0][None, :]
    xo_ref[...] = x_ref[...] + y


def _tail_xh_body(x_ref, parts_ref, w2_ref, b2_ref, wl_ref, bl_ref, w1n_ref,
                  xo_ref, xh_ref):
    _tail_body(x_ref, parts_ref, w2_ref, b2_ref, wl_ref, bl_ref, xo_ref)
    xh_ref[...] = _pack_bf16_pairs(
        lax.dot_general(xo_ref[...].astype(jnp.bfloat16),
                        w1n_ref[...].astype(jnp.bfloat16),
                        (((1,), (1,)), ((), ())),
                        preferred_element_type=jnp.float32))


def _tail(x, parts, w2l, b2l, wll, bll, w1next):
    common_in = [
        pl.BlockSpec((2000, H), lambda i: (i, 0)),
        pl.BlockSpec((2, 2000, F // 2), lambda i: (0, i, 0)),
        pl.BlockSpec((H, F), lambda i: (0, 0)),
        pl.BlockSpec((1, H), lambda i: (0, 0)),
        pl.BlockSpec((H, H), lambda i: (0, 0)),
        pl.BlockSpec((1, H), lambda i: (0, 0)),
    ]
    args = [x, parts, w2l, b2l.reshape(1, H), wll, bll.reshape(1, H)]
    if w1next is None:
        return pl.pallas_call(
            _tail_body,
            grid=(5,),
            in_specs=common_in,
            out_specs=pl.BlockSpec((2000, H), lambda i: (i, 0)),
            out_shape=jax.ShapeDtypeStruct((N, H), jnp.float32),
        )(*args)
    return pl.pallas_call(
        _tail_xh_body,
        grid=(5,),
        in_specs=common_in + [pl.BlockSpec((F, H), lambda i: (0, 0))],
        out_specs=[
            pl.BlockSpec((2000, H), lambda i: (i, 0)),
            pl.BlockSpec((2000, F // 2), lambda i: (i, 0)),
        ],
        out_shape=[
            jax.ShapeDtypeStruct((N, H), jnp.float32),
            jax.ShapeDtypeStruct((N, F // 2), jnp.int32),
        ],
    )(*args, w1next)


def _msg_body(t_hbm, srcr_hbm, nbrr_hbm, tidxr_hbm, xh_hbm, out_hbm,
              src_all, nbr_all, tidx_all, g_v0, g_v1, w_v0, w_v1,
              msg_v0, msg_v1, stage_v, agg_sh,
              sem_g0, sem_g1, sem_w0, sem_w1, sem_s0, sem_s1):
    c = lax.axis_index("c")
    s = lax.axis_index("s")
    wid = s * 2 + c
    zero16 = jnp.zeros((16,), jnp.float32)
    g_bufs = (g_v0, g_v1)
    w_bufs = (w_v0, w_v1)
    m_bufs = (msg_v0, msg_v1)
    g_sems = (sem_g0, sem_g1)
    w_sems = (sem_w0, sem_w1)
    s_sems = (sem_s0, sem_s1)

    pltpu.sync_copy(srcr_hbm.at[wid], src_all)
    pltpu.sync_copy(nbrr_hbm.at[wid], nbr_all)
    pltpu.sync_copy(tidxr_hbm.at[wid], tidx_all)

    def _zrow(r, _):
        for k in range(F // 16):
            msg_v0[r, pl.ds(k * 16, 16)] = zero16
        return 0
    lax.fori_loop(0, ZROWS, _zrow, 0)

    rbase = s * ROWS_PER_TILE

    def _zagg(r, _):
        pltpu.sync_copy(msg_v0.at[pl.ds(0, ZROWS)],
                        agg_sh.at[pl.ds(rbase + r * ZROWS, ZROWS)])
        return 0
    lax.fori_loop(0, ROWS_PER_TILE // ZROWS, _zagg, 0)
    plsc.subcore_barrier()

    def _start(ch, b):
        for k in range(SB // SUB):
            dst = pl.ds(k * SUB, SUB)
            pltpu.async_copy(xh_hbm.at[nbr_all.at[ch, k]],
                             g_bufs[b].at[dst], g_sems[b])
            pltpu.async_copy(t_hbm.at[tidx_all.at[ch, k]],
                             w_bufs[b].at[dst], w_sems[b])

    def _finish(ch, b, wait_prev):
        gb, wb, mb = g_bufs[b], w_bufs[b], m_bufs[b]
        pltpu.make_async_copy(xh_hbm.at[pl.ds(0, SB)], gb, g_sems[b]).wait()
        pltpu.make_async_copy(t_hbm.at[pl.ds(0, SB)], wb, w_sems[b]).wait()
        if wait_prev:
            for k in range(SB // SUB):
                pltpu.make_async_copy(mb.at[pl.ds(k * SUB, SUB)],
                                      agg_sh.at[src_all.at[ch, k]],
                                      s_sems[b]).wait()

        himask = jnp.full((16,), -65536, jnp.int32)
        sh16 = jnp.full((16,), 16, jnp.int32)

        def _mrow(r0, _):
            for rr in range(8):
                r = r0 * 8 + rr
                for k in range(F // 32):
                    sl = pl.ds(k * 16, 16)
                    gx = gb[r, sl]
                    wx = wb[r, sl]
                    g_lo = plsc.bitcast(lax.shift_left(gx, sh16), jnp.float32)
                    g_hi = plsc.bitcast(gx & himask, jnp.float32)
                    w_lo = plsc.bitcast(lax.shift_left(wx, sh16), jnp.float32)
                    w_hi = plsc.bitcast(wx & himask, jnp.float32)
                    mb[r, pl.ds(k * 16, 16)] = g_lo * w_lo
                    mb[r, pl.ds(F // 2 + k * 16, 16)] = g_hi * w_hi
            return 0
        lax.fori_loop(0, SB // 8, _mrow, 0)
        for k in range(SB // SUB):
            pltpu.async_copy(mb.at[pl.ds(k * SUB, SUB)],
                             agg_sh.at[src_all.at[ch, k]], s_sems[b],
                             add=True)

    _start(0, 0)
    _start(1, 1)
    _finish(0, 0, False)
    _start(2, 0)
    _finish(1, 1, False)
    _start(3, 1)

    def _pair(j0, _):
        ch = j0 * 2 + 2
        _finish(ch, 0, True)
        _start(ch + 2, 0)
        _finish(ch + 1, 1, True)
        _start(ch + 3, 1)
        return 0
    lax.fori_loop(0, (NSUPER - 4) // 2, _pair, 0)
    _finish(NSUPER - 2, 0, True)
    _finish(NSUPER - 1, 1, True)
    for b, ch in ((0, NSUPER - 2), (1, NSUPER - 1)):
        for k in range(SB // SUB):
            pltpu.make_async_copy(m_bufs[b].at[pl.ds(k * SUB, SUB)],
                                  agg_sh.at[src_all.at[ch, k]],
                                  s_sems[b]).wait()
    plsc.subcore_barrier()

    def _out(r, _):
        rows = pl.ds(rbase + r * ZROWS, ZROWS)
        pltpu.sync_copy(agg_sh.at[rows], msg_v0.at[pl.ds(0, ZROWS)])

        def _prow(q, _2):
            for k in range(F // 32):
                a = msg_v0[q, pl.ds(k * 16, 16)]
                bq = msg_v0[q, pl.ds(F // 2 + k * 16, 16)]
                packed = plsc.pack(a, bq, format=plsc.PackFormat.INTERLEAVED)
                stage_v[q, pl.ds(k * 16, 16)] = plsc.bitcast(packed, jnp.int32)
            return 0
        lax.fori_loop(0, ZROWS, _prow, 0)
        pltpu.sync_copy(stage_v.at[pl.ds(0, ZROWS)], out_hbm.at[c, rows])
        return 0
    lax.fori_loop(0, ROWS_PER_TILE // ZROWS, _out, 0)


def _messages(t_l, src_r, nbr_r, tidx_r, xh):
    mesh = plsc.VectorSubcoreMesh(core_axis_name="c", subcore_axis_name="s")
    kfn = functools.partial(
        pl.kernel,
        mesh=mesh,
        out_type=jax.ShapeDtypeStruct((2, NPAD, F // 2), jnp.int32),
        scratch_types=[
            pltpu.VMEM((NSUPER, SB // SUB, SUB), jnp.int32),
            pltpu.VMEM((NSUPER, SB // SUB, SUB), jnp.int32),
            pltpu.VMEM((NSUPER, SB // SUB, SUB), jnp.int32),
            pltpu.VMEM((SB, F // 2), jnp.int32),
            pltpu.VMEM((SB, F // 2), jnp.int32),
            pltpu.VMEM((SB, F // 2), jnp.int32),
            pltpu.VMEM((SB, F // 2), jnp.int32),
            pltpu.VMEM((SB, F), jnp.float32),
            pltpu.VMEM((SB, F), jnp.float32),
            pltpu.VMEM((ZROWS, F // 2), jnp.int32),
            pltpu.VMEM_SHARED((NPAD, F), jnp.float32),
            pltpu.SemaphoreType.DMA,
            pltpu.SemaphoreType.DMA,
            pltpu.SemaphoreType.DMA,
            pltpu.SemaphoreType.DMA,
            pltpu.SemaphoreType.DMA,
            pltpu.SemaphoreType.DMA,
        ],
        compiler_params=pltpu.CompilerParams(use_tc_tiling_on_sc=False,
                                             needs_layout_passes=False),
    )(_msg_body)
    return kfn(t_l, src_r, nbr_r, tidx_r, xh)


def kernel(z, edge_index, edge_weight, embedding, rbf_means, rbf_betas,
           mlp_w1, mlp_b1, mlp_w2, mlp_b2, lin1_w, lin2_w, lin2_b, lin_w,
           lin_b):
    src_r = edge_index[0].astype(jnp.int32).reshape(NTILE, NSUPER, SB // SUB, SUB)
    nbr_r = edge_index[1].astype(jnp.int32).reshape(NTILE, NSUPER, SB // SUB, SUB)


    x, xh = _embed(z, embedding, lin1_w[0])
    t_all = _tables(rbf_means, rbf_betas, mlp_w1, mlp_b1, mlp_w2, mlp_b2)
    tidx_r = _tidx(edge_weight).reshape(NTILE, NSUPER, SB // SUB, SUB)
    for l in range(L):
        parts = _messages(t_all[l], src_r, nbr_r, tidx_r, xh)
        if l < L - 1:
            x, xh = _tail(x, parts, lin2_w[l], lin2_b[l], lin_w[l], lin_b[l],
                          lin1_w[l + 1])
        else:
            x = _tail(x, parts, lin2_w[l], lin2_b[l], lin_w[l], lin_b[l], None)
    return x


# speedup vs baseline: 1.0515x; 1.0515x over previous
import functools

import jax
import jax.numpy as jnp
from jax import lax
from jax.experimental import pallas as pl
from jax.experimental.pallas import tpu as pltpu
from jax.experimental.pallas import tpu_sc as plsc

N = 10000
E = 320000
H = 128
F = 64
R = 50
L = 6
CUT = 5.0

NPAD = 10240
NTILE = 32
EPT = E // NTILE
B = 80
NCHUNK = EPT // B
ROWS_PER_TILE = NPAD // 16


def _pack_bf16_pairs(w):
    lo = w[:, :F // 2].astype(jnp.bfloat16).astype(jnp.float32)
    hi = w[:, F // 2:].astype(jnp.bfloat16).astype(jnp.float32)
    ulo = lax.shift_right_logical(lax.bitcast_convert_type(lo, jnp.int32), 16)
    uhi = lax.bitcast_convert_type(hi, jnp.int32) & jnp.int32(-65536)
    return ulo | uhi


def _embed_body(z_ref, emb_ref, w1_ref, x_ref, xh_ref):
    zv = z_ref[0, 0, :]
    col = lax.broadcasted_iota(jnp.int32, (zv.shape[0], H), 1)
    oh = (zv[:, None] == col).astype(jnp.float32)
    x = jnp.dot(oh, emb_ref[...], preferred_element_type=jnp.float32)
    x_ref[...] = x
    xh_ref[...] = _pack_bf16_pairs(
        lax.dot_general(x.astype(jnp.bfloat16),
                        w1_ref[...].astype(jnp.bfloat16),
                        (((1,), (1,)), ((), ())),
                        preferred_element_type=jnp.float32))


def _embed(z, embedding, w1l):
    z3 = z.astype(jnp.int32).reshape(10, 1, N // 10)
    emb_pad = jnp.zeros((H, H), jnp.float32).at[:100, :].set(embedding)
    return pl.pallas_call(
        _embed_body,
        grid=(10,),
        in_specs=[
            pl.BlockSpec((1, 1, N // 10), lambda i: (i, 0, 0)),
            pl.BlockSpec((H, H), lambda i: (0, 0)),
            pl.BlockSpec((F, H), lambda i: (0, 0)),
        ],
        out_specs=[
            pl.BlockSpec((N // 10, H), lambda i: (i, 0)),
            pl.BlockSpec((N // 10, F // 2), lambda i: (i, 0)),
        ],
        out_shape=[
            jax.ShapeDtypeStruct((N, H), jnp.float32),
            jax.ShapeDtypeStruct((N, F // 2), jnp.int32),
        ],
    )(z3, emb_pad, w1l)


KTAB = 16384
CT = 4096


def _table_body(means_ref, betas_ref, w1_ref, b1_ref, w2_ref, b2_ref,
                out_ref):
    base = pl.program_id(1) * CT
    j = lax.broadcasted_iota(jnp.int32, (CT,), 0) + base
    d = j.astype(jnp.float32) * (CUT / KTAB)
    t = jnp.exp(-d)
    m = means_ref[0]
    be = betas_ref[0]
    ea = jnp.exp(-be[None, :] * (t[:, None] - m[None, :]) ** 2)
    h = lax.dot_general(ea.astype(jnp.bfloat16), w1_ref[0].astype(jnp.bfloat16),
                        (((1,), (1,)), ((), ())),
                        preferred_element_type=jnp.float32)
    h = h + b1_ref[0, 0][None, :]
    h = h * jax.nn.sigmoid(h)
    w = lax.dot_general(h.astype(jnp.bfloat16), w2_ref[0].astype(jnp.bfloat16),
                        (((1,), (1,)), ((), ())),
                        preferred_element_type=jnp.float32)
    w = w + b2_ref[0, 0][None, :]
    c = 0.5 * (jnp.cos(d * (jnp.pi / CUT)) + 1.0) * (d < CUT).astype(jnp.float32)
    out_ref[0] = _pack_bf16_pairs(w * c[:, None])


def _tables(rbf_means, rbf_betas, mlp_w1, mlp_b1, mlp_w2, mlp_b2):
    return pl.pallas_call(
        _table_body,
        grid=(L, KTAB // CT),
        in_specs=[
            pl.BlockSpec((1, R), lambda l, i: (0, 0)),
            pl.BlockSpec((1, R), lambda l, i: (0, 0)),
            pl.BlockSpec((1, F, R), lambda l, i: (l, 0, 0)),
            pl.BlockSpec((1, 1, F), lambda l, i: (l, 0, 0)),
            pl.BlockSpec((1, F, F), lambda l, i: (l, 0, 0)),
            pl.BlockSpec((1, 1, F), lambda l, i: (l, 0, 0)),
        ],
        out_specs=pl.BlockSpec((1, CT, F // 2), lambda l, i: (l, i, 0)),
        out_shape=jax.ShapeDtypeStruct((L, KTAB, F // 2), jnp.int32),
    )(rbf_means.reshape(1, R), rbf_betas.reshape(1, R),
      mlp_w1, mlp_b1.reshape(L, 1, F), mlp_w2, mlp_b2.reshape(L, 1, F))


NB = 100
CE = E // NB


def _tidx_body(ew_ref, out_ref):
    d = ew_ref[0, 0, :]
    out_ref[0, 0, :] = (d * (KTAB / CUT) + 0.5).astype(jnp.int32)


def _tidx(edge_weight):
    return pl.pallas_call(
        _tidx_body,
        grid=(NB,),
        in_specs=[pl.BlockSpec((1, 1, CE), lambda i: (i, 0, 0))],
        out_specs=pl.BlockSpec((1, 1, CE), lambda i: (i, 0, 0)),
        out_shape=jax.ShapeDtypeStruct((NB, 1, CE), jnp.int32),
    )(edge_weight.reshape(NB, 1, CE))


def _tail_body(x_ref, parts_ref, w2_ref, b2_ref, wl_ref, bl_ref, xo_ref):
    agg = parts_ref[0] + parts_ref[1]
    y = lax.dot_general(agg.astype(jnp.bfloat16),
                        w2_ref[...].astype(jnp.bfloat16),
                        (((1,), (1,)), ((), ())),
                        preferred_element_type=jnp.float32)
    y = y + b2_ref[0][None, :]
    y = y * jax.nn.sigmoid(y)
    y = lax.dot_general(y.astype(jnp.bfloat16),
                        wl_ref[...].astype(jnp.bfloat16),
                        (((1,), (1,)), ((), ())),
                        preferred_element_type=jnp.float32)
    y = y + bl_ref[0][None, :]
    xo_ref[...] = x_ref[...] + y


def _tail_xh_body(x_ref, parts_ref, w2_ref, b2_ref, wl_ref, bl_ref, w1n_ref,
                  xo_ref, xh_ref):
    _tail_body(x_ref, parts_ref, w2_ref, b2_ref, wl_ref, bl_ref, xo_ref)
    xh_ref[...] = _pack_bf16_pairs(
        lax.dot_general(xo_ref[...].astype(jnp.bfloat16),
                        w1n_ref[...].astype(jnp.bfloat16),
                        (((1,), (1,)), ((), ())),
                        preferred_element_type=jnp.float32))


def _tail(x, parts, w2l, b2l, wll, bll, w1next):
    common_in = [
        pl.BlockSpec((2000, H), lambda i: (i, 0)),
        pl.BlockSpec((2, 2000, F), lambda i: (0, i, 0)),
        pl.BlockSpec((H, F), lambda i: (0, 0)),
        pl.BlockSpec((1, H), lambda i: (0, 0)),
        pl.BlockSpec((H, H), lambda i: (0, 0)),
        pl.BlockSpec((1, H), lambda i: (0, 0)),
    ]
    args = [x, parts, w2l, b2l.reshape(1, H), wll, bll.reshape(1, H)]
    if w1next is None:
        return pl.pallas_call(
            _tail_body,
            grid=(5,),
            in_specs=common_in,
            out_specs=pl.BlockSpec((2000, H), lambda i: (i, 0)),
            out_shape=jax.ShapeDtypeStruct((N, H), jnp.float32),
        )(*args)
    return pl.pallas_call(
        _tail_xh_body,
        grid=(5,),
        in_specs=common_in + [pl.BlockSpec((F, H), lambda i: (0, 0))],
        out_specs=[
            pl.BlockSpec((2000, H), lambda i: (i, 0)),
            pl.BlockSpec((2000, F // 2), lambda i: (i, 0)),
        ],
        out_shape=[
            jax.ShapeDtypeStruct((N, H), jnp.float32),
            jax.ShapeDtypeStruct((N, F // 2), jnp.int32),
        ],
    )(*args, w1next)


def _msg_body(t_hbm, srcr_hbm, nbrr_hbm, tidxr_hbm, xh_hbm, out_hbm,
              src_all, nbr_all, tidx_all, g_v0, g_v1, w_v0, w_v1,
              msg_v0, msg_v1, agg_sh,
              sem_g0, sem_g1, sem_w0, sem_w1, sem_s0, sem_s1):
    c = lax.axis_index("c")
    s = lax.axis_index("s")
    wid = s * 2 + c
    zero16 = jnp.zeros((16,), jnp.float32)
    g_bufs = (g_v0, g_v1)
    w_bufs = (w_v0, w_v1)
    m_bufs = (msg_v0, msg_v1)
    g_sems = (sem_g0, sem_g1)
    w_sems = (sem_w0, sem_w1)
    s_sems = (sem_s0, sem_s1)

    pltpu.sync_copy(srcr_hbm.at[wid], src_all)
    pltpu.sync_copy(nbrr_hbm.at[wid], nbr_all)
    pltpu.sync_copy(tidxr_hbm.at[wid], tidx_all)

    def _zrow(r, _):
        for k in range(F // 16):
            msg_v0[r, pl.ds(k * 16, 16)] = zero16
        return 0
    lax.fori_loop(0, B, _zrow, 0)

    rbase = s * ROWS_PER_TILE

    def _zagg(r, _):
        pltpu.sync_copy(msg_v0, agg_sh.at[pl.ds(rbase + r * B, B)])
        return 0
    lax.fori_loop(0, ROWS_PER_TILE // B, _zagg, 0)
    plsc.subcore_barrier()

    def _start(ch, b):
        pltpu.async_copy(xh_hbm.at[nbr_all.at[ch]], g_bufs[b], g_sems[b])
        pltpu.async_copy(t_hbm.at[tidx_all.at[ch]], w_bufs[b], w_sems[b])

    def _finish(ch, b, wait_prev):
        gb, wb, mb = g_bufs[b], w_bufs[b], m_bufs[b]
        pltpu.make_async_copy(xh_hbm.at[nbr_all.at[ch]], gb, g_sems[b]).wait()
        pltpu.make_async_copy(t_hbm.at[tidx_all.at[ch]], wb, w_sems[b]).wait()
        if wait_prev:
            pltpu.make_async_copy(mb, agg_sh.at[src_all.at[ch]],
                                  s_sems[b]).wait()

        himask = jnp.full((16,), -65536, jnp.int32)
        sh16 = jnp.full((16,), 16, jnp.int32)

        def _mrow(r0, _):
            for rr in range(4):
                r = r0 * 4 + rr
                for k in range(F // 32):
                    sl = pl.ds(k * 16, 16)
                    gx = gb[r, sl]
                    wx = wb[r, sl]
                    g_lo = plsc.bitcast(lax.shift_left(gx, sh16), jnp.float32)
                    g_hi = plsc.bitcast(gx & himask, jnp.float32)
                    w_lo = plsc.bitcast(lax.shift_left(wx, sh16), jnp.float32)
                    w_hi = plsc.bitcast(wx & himask, jnp.float32)
                    mb[r, pl.ds(k * 16, 16)] = g_lo * w_lo
                    mb[r, pl.ds(F // 2 + k * 16, 16)] = g_hi * w_hi
            return 0
        lax.fori_loop(0, B // 4, _mrow, 0)
        pltpu.async_copy(mb, agg_sh.at[src_all.at[ch]], s_sems[b], add=True)

    _start(0, 0)
    _start(1, 1)
    _finish(0, 0, False)
    _start(2, 0)
    _finish(1, 1, False)
    _start(3, 1)

    def _pair(j0, _):
        ch = j0 * 2 + 2
        _finish(ch, 0, True)
        _start(ch + 2, 0)
        _finish(ch + 1, 1, True)
        _start(ch + 3, 1)
        return 0
    lax.fori_loop(0, (NCHUNK - 5) // 2, _pair, 0)
    _finish(NCHUNK - 3, 0, True)
    _start(NCHUNK - 1, 0)
    _finish(NCHUNK - 2, 1, True)
    _finish(NCHUNK - 1, 0, True)
    pltpu.make_async_copy(msg_v1, agg_sh.at[src_all.at[NCHUNK - 2]],
                          sem_s1).wait()
    pltpu.make_async_copy(msg_v0, agg_sh.at[src_all.at[NCHUNK - 1]],
                          sem_s0).wait()
    plsc.subcore_barrier()

    def _out(r, _):
        rows = pl.ds(rbase + r * B, B)
        pltpu.sync_copy(agg_sh.at[rows], msg_v0)
        pltpu.sync_copy(msg_v0, out_hbm.at[c, rows])
        return 0
    lax.fori_loop(0, ROWS_PER_TILE // B, _out, 0)


def _messages(t_l, src_r, nbr_r, tidx_r, xh):
    mesh = plsc.VectorSubcoreMesh(core_axis_name="c", subcore_axis_name="s")
    kfn = functools.partial(
        pl.kernel,
        mesh=mesh,
        out_type=jax.ShapeDtypeStruct((2, NPAD, F), jnp.float32),
        scratch_types=[
            pltpu.VMEM((NCHUNK, B), jnp.int32),
            pltpu.VMEM((NCHUNK, B), jnp.int32),
            pltpu.VMEM((NCHUNK, B), jnp.int32),
            pltpu.VMEM((B, F // 2), jnp.int32),
            pltpu.VMEM((B, F // 2), jnp.int32),
            pltpu.VMEM((B, F // 2), jnp.int32),
            pltpu.VMEM((B, F // 2), jnp.int32),
            pltpu.VMEM((B, F), jnp.float32),
            pltpu.VMEM((B, F), jnp.float32),
            pltpu.VMEM_SHARED((NPAD, F), jnp.float32),
            pltpu.SemaphoreType.DMA,
            pltpu.SemaphoreType.DMA,
            pltpu.SemaphoreType.DMA,
            pltpu.SemaphoreType.DMA,
            pltpu.SemaphoreType.DMA,
            pltpu.SemaphoreType.DMA,
        ],
        compiler_params=pltpu.CompilerParams(use_tc_tiling_on_sc=False,
                                             needs_layout_passes=False),
    )(_msg_body)
    return kfn(t_l, src_r, nbr_r, tidx_r, xh)


def kernel(z, edge_index, edge_weight, embedding, rbf_means, rbf_betas,
           mlp_w1, mlp_b1, mlp_w2, mlp_b2, lin1_w, lin2_w, lin2_b, lin_w,
           lin_b):
    src_r = edge_index[0].astype(jnp.int32).reshape(NTILE, NCHUNK, B)
    nbr_r = edge_index[1].astype(jnp.int32).reshape(NTILE, NCHUNK, B)


    x, xh = _embed(z, embedding, lin1_w[0])
    t_all = _tables(rbf_means, rbf_betas, mlp_w1, mlp_b1, mlp_w2, mlp_b2)
    tidx_r = _tidx(edge_weight).reshape(NTILE, NCHUNK, B)
    for l in range(L):
        parts = _messages(t_all[l], src_r, nbr_r, tidx_r, xh)
        if l < L - 1:
            x, xh = _tail(x, parts, lin2_w[l], lin2_b[l], lin_w[l], lin_b[l],
                          lin1_w[l + 1])
        else:
            x = _tail(x, parts, lin2_w[l], lin2_b[l], lin_w[l], lin_b[l], None)
    return x
